# baseline (speedup 1.0000x reference)
import functools

import jax
import jax.numpy as jnp
from jax import lax
from jax.experimental import pallas as pl
from jax.experimental.pallas import tpu as pltpu
from jax.experimental.pallas import tpu_sc as plsc

_B = 16384
_D = 64
_AR = 20
_NN = 5
_NG = _NN + 1
_NC = 2
_NS = 16
_NW = _NC * _NS
_SPW = _B // _NW
_CS = 2
_NCH = _SPW // _CS
_GPC = _CS * _NG
_RPC = _GPC * _AR
_WROWS = _NG * _AR
_NWIN = _RPC // _WROWS
_LANES = 16
_NQ = _D // _LANES
_WPROW = _SPW * _NG * _LANES // 128


def _sc_partials(xp_flat, xn_flat, emb_pairs):
    mesh = plsc.VectorSubcoreMesh(core_axis_name="c", subcore_axis_name="s")

    @functools.partial(
        pl.kernel,
        out_type=jax.ShapeDtypeStruct((_B * _NG * _LANES // 128, 128),
                                      jnp.float32),
        mesh=mesh,
        compiler_params=pltpu.CompilerParams(use_tc_tiling_on_sc=True),
        scratch_types=[
            pltpu.VMEM((_RPC,), jnp.int32),
            pltpu.VMEM((_RPC,), jnp.int32),
            pltpu.VMEM((_RPC,), jnp.int32),
            pltpu.VMEM((_RPC + _LANES,), jnp.int32),
            pltpu.VMEM((_RPC + _LANES,), jnp.int32),
            pltpu.VMEM((_RPC, 2 * _D), jnp.float32),
            pltpu.VMEM((_RPC, 2 * _D), jnp.float32),
            pltpu.VMEM((_WPROW, 128), jnp.float32),
            pltpu.SemaphoreType.DMA,
            pltpu.SemaphoreType.DMA,
        ],
    )
    def k(xp_hbm, xn_hbm, emb_hbm, out_hbm,
          raw_v, pidx_v0, pidx_v1, hoff_v0, hoff_v1, rows_v0, rows_v1,
          part_v, sem0, sem1):
        wid = lax.axis_index("s") * _NC + lax.axis_index("c")
        pidx_b = (pidx_v0, pidx_v1)
        hoff_b = (hoff_v0, hoff_v1)
        rows_b = (rows_v0, rows_v1)
        sem_b = (sem0, sem1)

        def fire(ch, b):
            pltpu.sync_copy(
                xp_hbm.at[pl.ds(wid * (_SPW * _AR) + ch * (_CS * _AR), _CS * _AR)],
                raw_v.at[pl.ds(0, _CS * _AR)])
            pltpu.sync_copy(
                xn_hbm.at[pl.ds(wid * (_SPW * _NN * _AR) + ch * (_CS * _NN * _AR),
                                _CS * _NN * _AR)],
                raw_v.at[pl.ds(_CS * _AR, _CS * _NN * _AR)])
            for t in range(_RPC // _LANES):
                v = raw_v[pl.ds(t * _LANES, _LANES)]
                pidx_b[b][pl.ds(t * _LANES, _LANES)] = v >> 1
                hoff_b[b][pl.ds(t * _LANES, _LANES)] = (v & 1) * _D
            for j in range(_NWIN):
                pltpu.async_copy(
                    emb_hbm.at[pidx_b[b].at[pl.ds(j * _WROWS, _WROWS)]],
                    rows_b[b].at[pl.ds(j * _WROWS, _WROWS)],
                    sem_b[b])

        def drain_rows(b):
            pltpu.make_async_copy(
                emb_hbm.at[pl.ds(0, _RPC)], rows_b[b], sem_b[b]).wait()

        fire(0, 0)
        fire(1, 1)

        @pl.loop(0, _NCH // 2)
        def _pair(p):
            for b in range(2):
                ch = 2 * p + b
                drain_rows(b)

                @pl.when(ch < _NCH - 2)
                def _():
                    fire(ch + 2, b)

                @pl.loop(0, _CS)
                def _sample(sl):
                    for g in range(_NG):
                        if g == 0:
                            base = sl * _AR
                        else:
                            base = _CS * _AR + sl * (_NN * _AR) + (g - 1) * _AR
                        off0 = hoff_b[b][pl.ds(base, _LANES)][0]
                        acc = [rows_b[b][pl.ds(base, 1),
                                         pl.ds(off0 + q * _LANES, _LANES)]
                               for q in range(_NQ)]
                        for r in range(1, _AR):
                            offr = hoff_b[b][pl.ds(base + r, _LANES)][0]
                            for q in range(_NQ):
                                acc[q] += rows_b[b][pl.ds(base + r, 1),
                                                    pl.ds(offr + q * _LANES,
                                                          _LANES)]
                        sq = acc[0] * acc[0]
                        for q in range(1, _NQ):
                            sq += acc[q] * acc[q]
                        gi = ch * _GPC + sl * _NG + g
                        part_v[pl.ds(gi // 8, 1),
                               pl.ds((gi % 8) * _LANES, _LANES)] = sq

        pltpu.sync_copy(part_v, out_hbm.at[pl.ds(wid * _WPROW, _WPROW), :])

    return k(xp_flat, xn_flat, emb_pairs)


def _tc_score(parts):
    def body(p_ref, o_ref):
        v = p_ref[...]
        for sh in (1, 2, 4, 8):
            v = v + jnp.concatenate([v[:, sh:], v[:, :sh]], axis=1)
        r = lax.broadcasted_iota(jnp.int32, v.shape, 0)
        c = lax.broadcasted_iota(jnp.int32, v.shape, 1)
        gi = r * (v.shape[1] // _LANES) + c // _LANES
        is_start = (c % _LANES) == 0
        y = jnp.where((gi % _NG) == 0, v, jnp.reciprocal(v))
        val = jnp.log(jnp.tanh(y))
        val = jnp.where(is_start, val, 0.0)
        o_ref[...] = (jnp.sum(val) * (1.0 / _B)).reshape(1, 1)

    return pl.pallas_call(
        body,
        out_shape=jax.ShapeDtypeStruct((1, 1), jnp.float32),
    )(parts)


def kernel(x_pos, x_neg, emb):
    xp = x_pos.reshape(-1)
    xn = x_neg.reshape(-1)
    emb_pairs = emb.reshape(emb.shape[0] // 2, 2 * emb.shape[1])
    parts = _sc_partials(xp, xn, emb_pairs)
    return _tc_score(parts).reshape(())
